# baseline (speedup 1.0000x reference)
import jax
import jax.numpy as jnp
from jax import lax
from jax.experimental import pallas as pl
from jax.experimental.pallas import tpu as pltpu
from jax.experimental.pallas import tpu_sc as plsc

N0, N1, N2 = 100000, 20000, 4096
D = 128
E0, E1 = 320000, 65536
BN_EPS = 1e-5

NC, NS, L = 2, 16, 16
CH = 128

HALF0 = N1 // NC
DUMMY0 = HALF0
ACC0 = 10008
ROWS0 = E0 // CH
ITER0 = (ROWS0 + NS - 1) // NS

ACC1 = N2
ROWS1 = E1 // CH
ITER1 = ROWS1 // (NC * NS)


def _make_agg_body(rowfn, validfn, remapfn, flushfn, zerofn, iters):
    jmax = ((iters + 5) // 6) * 6

    def body(x_hbm, src_hbm, dst_hbm, zeros_hbm, out_hbm, *s):
        cid = lax.axis_index("c")
        sid = lax.axis_index("s")
        sidx = s[0:3]
        draw = s[3:6]
        didx = s[6:8]
        rows = s[8:11]
        acc = s[11]
        semi = s[12:15]
        semg = s[15:18]
        sems = s[18:20]

        def v(j):
            return (j >= 0) & validfn(cid, sid, j)

        def start_idx(j, p):
            @pl.when(v(j))
            def _():
                r = rowfn(cid, sid, j)
                pltpu.async_copy(src_hbm.at[r], sidx[p], semi[p])
                pltpu.async_copy(dst_hbm.at[r], draw[p], semi[p])

        def wait_idx(j, p):
            @pl.when(v(j))
            def _():
                r = rowfn(cid, sid, j)
                pltpu.make_async_copy(src_hbm.at[r], sidx[p], semi[p]).wait()
                pltpu.make_async_copy(dst_hbm.at[r], draw[p], semi[p]).wait()

        def start_gather(j, p):
            @pl.when(v(j))
            def _():
                pltpu.async_copy(x_hbm.at[sidx[p]], rows[p], semg[p])

        def wait_scatter(j, p, q):
            @pl.when(v(j))
            def _():
                pltpu.make_async_copy(
                    rows[p], acc.at[didx[q].at[0]], sems[q]).wait()

        def step(j, p, q, p2):

            @pl.when(v(j))
            def _():
                pltpu.make_async_copy(x_hbm.at[sidx[p]], rows[p], semg[p]).wait()
                for k in range(CH // L):
                    d = draw[p][pl.ds(k * L, L)]
                    didx[q][0, pl.ds(k * L, L)] = remapfn(cid, d)
                pltpu.async_copy(rows[p], acc.at[didx[q].at[0]], sems[q],
                                 add=True)

            start_idx(j + 3, p)
            wait_scatter(j - 1, p2, 1 - q)
            wait_idx(j + 2, p2)
            start_gather(j + 2, p2)

        for u in range(3):
            start_idx(u, u)
        zerofn(sid, acc, zeros_hbm)
        for u in range(2):
            wait_idx(u, u)
            start_gather(u, u)
        plsc.subcore_barrier()

        def loop(ii, carry):
            for u in range(6):
                step(6 * ii + u, u % 3, u % 2, (u + 2) % 3)
            return carry

        lax.fori_loop(0, jmax // 6, loop, 0)
        wait_scatter(jmax - 1, (jmax - 1) % 3, (jmax - 1) % 2)
        plsc.subcore_barrier()
        flushfn(cid, sid, acc, out_hbm)

    return body


def _agg_kernel(body, out_rows, acc_rows):
    mesh = plsc.VectorSubcoreMesh(core_axis_name="c", subcore_axis_name="s")
    return pl.kernel(
        body,
        out_type=jax.ShapeDtypeStruct((out_rows, D), jnp.float32),
        mesh=mesh,
        scratch_types=(
            [pltpu.VMEM((CH,), jnp.int32)] * 3
            + [pltpu.VMEM((CH,), jnp.int32)] * 3
            + [pltpu.VMEM((1, CH), jnp.int32)] * 2
            + [pltpu.VMEM((CH, D), jnp.float32)] * 3
            + [pltpu.VMEM_SHARED((acc_rows, D), jnp.float32)]
            + [pltpu.SemaphoreType.DMA] * 8
        ),
    )


def _remap0(cid, d):
    local = d - cid * HALF0
    oob = (local < 0) | (local >= HALF0)
    return jnp.where(oob, DUMMY0, local)


def _zero0(sid, acc, zeros_hbm):
    pltpu.sync_copy(zeros_hbm, acc.at[pl.ds(sid * 624, 624)])

    @pl.when(sid == 0)
    def _():
        pltpu.sync_copy(zeros_hbm.at[pl.ds(0, 24)], acc.at[pl.ds(9984, 24)])


def _zero1(sid, acc, zeros_hbm):
    pltpu.sync_copy(zeros_hbm, acc.at[pl.ds(sid * (ACC1 // NS), ACC1 // NS)])


def _flush0(cid, sid, acc, out_hbm):
    @pl.when(sid < NS - 1)
    def _():
        pltpu.sync_copy(acc.at[pl.ds(sid * 632, 632)],
                        out_hbm.at[pl.ds(cid * HALF0 + sid * 632, 632)])

    @pl.when(sid == NS - 1)
    def _():
        pltpu.sync_copy(acc.at[pl.ds((NS - 1) * 632, 520)],
                        out_hbm.at[pl.ds(cid * HALF0 + (NS - 1) * 632, 520)])


def _flush1(cid, sid, acc, out_hbm):
    frows = ACC1 // NS
    pltpu.sync_copy(acc.at[pl.ds(sid * frows, frows)],
                    out_hbm.at[pl.ds(cid * ACC1 + sid * frows, frows)])


_agg0_body = _make_agg_body(
    rowfn=lambda cid, sid, j: sid + NS * j,
    validfn=lambda cid, sid, j: sid + NS * j < ROWS0,
    remapfn=_remap0,
    flushfn=_flush0,
    zerofn=_zero0,
    iters=ITER0,
)

_agg1_body = _make_agg_body(
    rowfn=lambda cid, sid, j: cid * (ROWS1 // NC) + sid + NS * j,
    validfn=lambda cid, sid, j: j < ITER1,
    remapfn=lambda cid, d: d,
    flushfn=_flush1,
    zerofn=_zero1,
    iters=ITER1,
)


def _bdot(a, w_ref):
    return jnp.dot(a.astype(jnp.bfloat16), w_ref[...],
                   preferred_element_type=jnp.float32)


def _mlp_body(a_ref, x_ref, w1_ref, b1_ref, w2_ref, b2_ref, o_ref):
    h = a_ref[...] + x_ref[...]
    h = jnp.maximum(_bdot(h, w1_ref) + b1_ref[...], 0.0)
    o_ref[...] = jnp.maximum(_bdot(h, w2_ref) + b2_ref[...], 0.0)


def _head_body(p0_ref, p1_ref, x_ref, w1_ref, b1_ref, w2_ref, b2_ref,
               l1_ref, c1_ref, l2_ref, c2_ref, o_ref):
    h = p0_ref[...] + p1_ref[...] + x_ref[...]
    h = jnp.maximum(_bdot(h, w1_ref) + b1_ref[...], 0.0)
    h = jnp.maximum(_bdot(h, w2_ref) + b2_ref[...], 0.0)
    h = jnp.maximum(_bdot(h, l1_ref) + c1_ref[...], 0.0)
    z = _bdot(h, l2_ref) + c2_ref[...]
    m = jnp.max(z, axis=-1, keepdims=True)
    e = jnp.exp(z - m)
    s = jnp.sum(e, axis=-1, keepdims=True)
    o_ref[...] = z - m - jnp.log(s)


def _full(shape):
    return pl.BlockSpec(shape, lambda i: (0, 0))


def _row(block, width=D, off=0):
    return pl.BlockSpec((block, width), lambda i, o=off: (i + o, 0))


def _mlp(aggr, x_full, w1, b1, w2, b2, rows, block):
    return pl.pallas_call(
        _mlp_body,
        grid=(rows // block,),
        in_specs=[_row(block), _row(block), _full((D, D)), _full((1, D)),
                  _full((D, D)), _full((1, D))],
        out_specs=_row(block),
        out_shape=jax.ShapeDtypeStruct((rows, D), jnp.float32),
    )(aggr, x_full, w1, b1, w2, b2)


def _head(parts, h_full, w1, b1, w2, b2, l1, c1, l2, c2, rows, block, dout):
    return pl.pallas_call(
        _head_body,
        grid=(rows // block,),
        in_specs=[_row(block), _row(block, off=N2 // block), _row(block),
                  _full((D, D)), _full((1, D)), _full((D, D)), _full((1, D)),
                  _full((D, D)), _full((1, D)), _full((D, dout)), _full((1, dout))],
        out_specs=pl.BlockSpec((block, dout), lambda i: (i, 0)),
        out_shape=jax.ShapeDtypeStruct((rows, dout), jnp.float32),
    )(parts, parts, h_full, w1, b1, w2, b2, l1, c1, l2, c2)


@jax.jit
def kernel(x, src0, dst0, src1, dst1, W0a, b0a, bn0_w, bn0_b, W0b, b0b,
           W1a, b1a, bn1_w, bn1_b, W1b, b1b, lin1_w, lin1_b, lin2_w, lin2_b):
    x = x.astype(jnp.float32)
    src0_2d = src0.astype(jnp.int32).reshape(ROWS0, CH)
    dst0_2d = dst0.astype(jnp.int32).reshape(ROWS0, CH)
    src1_2d = src1.astype(jnp.int32).reshape(ROWS1, CH)
    dst1_2d = dst1.astype(jnp.int32).reshape(ROWS1, CH)

    zeros0 = jnp.zeros((624, D), jnp.float32)
    zeros1 = jnp.zeros((ACC1 // NS, D), jnp.float32)

    s0 = bn0_w / jnp.sqrt(1.0 + BN_EPS)
    w0a = (W0a.T * s0[None, :]).astype(jnp.bfloat16)
    c0a = (b0a * s0 + bn0_b).reshape(1, D)
    s1 = bn1_w / jnp.sqrt(1.0 + BN_EPS)
    w1a = (W1a.T * s1[None, :]).astype(jnp.bfloat16)
    c1a = (b1a * s1 + bn1_b).reshape(1, D)
    dout = lin2_w.shape[0]

    agg0 = _agg_kernel(_agg0_body, N1, ACC0)
    aggr0 = agg0(x, src0_2d, dst0_2d, zeros0)
    h = _mlp(aggr0, x, w0a, c0a, W0b.T.astype(jnp.bfloat16),
             b0b.reshape(1, D), N1, 2000)

    agg1 = _agg_kernel(_agg1_body, NC * N2, ACC1)
    parts = agg1(h, src1_2d, dst1_2d, zeros1)
    out = _head(parts, h, w1a, c1a, W1b.T.astype(jnp.bfloat16),
                b1b.reshape(1, D), lin1_w.T.astype(jnp.bfloat16),
                lin1_b.reshape(1, D), lin2_w.T.astype(jnp.bfloat16),
                lin2_b.reshape(1, dout), N2, 1024, dout)
    return out


# speedup vs baseline: 1.0044x; 1.0044x over previous
import jax
import jax.numpy as jnp
from jax import lax
from jax.experimental import pallas as pl
from jax.experimental.pallas import tpu as pltpu
from jax.experimental.pallas import tpu_sc as plsc

N0, N1, N2 = 100000, 20000, 4096
D = 128
E0, E1 = 320000, 65536
BN_EPS = 1e-5

NC, NS, L = 2, 16, 16
CH = 128

HALF0 = N1 // NC
DUMMY0 = HALF0
ACC0 = 10008
ROWS0 = E0 // CH
ITER0 = (ROWS0 + NS - 1) // NS

ACC1 = N2
ROWS1 = E1 // CH
ITER1 = ROWS1 // (NC * NS)


def _make_agg_body(rowfn, validfn, remapfn, flushfn, zerofn, iters):
    jmax = ((iters + 5) // 6) * 6

    def body(x_hbm, src_hbm, dst_hbm, zeros_hbm, out_hbm, *s):
        cid = lax.axis_index("c")
        sid = lax.axis_index("s")
        sidx = s[0:3]
        draw = s[3:6]
        didx = s[6:8]
        rows = s[8:11]
        acc = s[11]
        semi = s[12:15]
        semg = s[15:18]
        sems = s[18:20]

        def v(j):
            return (j >= 0) & validfn(cid, sid, j)

        def start_idx(j, p):
            @pl.when(v(j))
            def _():
                r = rowfn(cid, sid, j)
                pltpu.async_copy(src_hbm.at[r], sidx[p], semi[p])
                pltpu.async_copy(dst_hbm.at[r], draw[p], semi[p])

        def wait_idx(j, p):
            @pl.when(v(j))
            def _():
                r = rowfn(cid, sid, j)
                pltpu.make_async_copy(src_hbm.at[r], sidx[p], semi[p]).wait()
                pltpu.make_async_copy(dst_hbm.at[r], draw[p], semi[p]).wait()

        def start_gather(j, p):
            @pl.when(v(j))
            def _():
                pltpu.async_copy(x_hbm.at[sidx[p]], rows[p], semg[p])

        def wait_scatter(j, p, q):
            @pl.when(v(j))
            def _():
                pltpu.make_async_copy(
                    rows[p], acc.at[didx[q].at[0]], sems[q]).wait()

        def step(j, p, q, p2):

            @pl.when(v(j))
            def _():
                pltpu.make_async_copy(x_hbm.at[sidx[p]], rows[p], semg[p]).wait()
                for k in range(CH // L):
                    d = draw[p][pl.ds(k * L, L)]
                    didx[q][0, pl.ds(k * L, L)] = remapfn(cid, d)
                pltpu.async_copy(rows[p], acc.at[didx[q].at[0]], sems[q],
                                 add=True)

            start_idx(j + 3, p)
            wait_scatter(j - 1, p2, 1 - q)
            wait_idx(j + 2, p2)
            start_gather(j + 2, p2)

        for u in range(3):
            start_idx(u, u)
        zerofn(sid, acc, zeros_hbm)
        for u in range(2):
            wait_idx(u, u)
            start_gather(u, u)
        plsc.subcore_barrier()

        def loop(ii, carry):
            for u in range(6):
                step(6 * ii + u, u % 3, u % 2, (u + 2) % 3)
            return carry

        lax.fori_loop(0, jmax // 6, loop, 0)
        wait_scatter(jmax - 1, (jmax - 1) % 3, (jmax - 1) % 2)
        plsc.subcore_barrier()
        flushfn(cid, sid, acc, out_hbm)

    return body


def _agg_kernel(body, out_rows, acc_rows):
    mesh = plsc.VectorSubcoreMesh(core_axis_name="c", subcore_axis_name="s")
    return pl.kernel(
        body,
        out_type=jax.ShapeDtypeStruct((out_rows, D), jnp.float32),
        mesh=mesh,
        scratch_types=(
            [pltpu.VMEM((CH,), jnp.int32)] * 3
            + [pltpu.VMEM((CH,), jnp.int32)] * 3
            + [pltpu.VMEM((1, CH), jnp.int32)] * 2
            + [pltpu.VMEM((CH, D), jnp.float32)] * 3
            + [pltpu.VMEM_SHARED((acc_rows, D), jnp.float32)]
            + [pltpu.SemaphoreType.DMA] * 8
        ),
    )


def _remap0(cid, d):
    local = d - cid * HALF0
    oob = (local < 0) | (local >= HALF0)
    return jnp.where(oob, DUMMY0, local)


def _zero0(sid, acc, zeros_hbm):
    pltpu.sync_copy(zeros_hbm, acc.at[pl.ds(sid * 624, 624)])

    @pl.when(sid == 0)
    def _():
        pltpu.sync_copy(zeros_hbm.at[pl.ds(0, 24)], acc.at[pl.ds(9984, 24)])


def _zero1(sid, acc, zeros_hbm):
    pltpu.sync_copy(zeros_hbm, acc.at[pl.ds(sid * (ACC1 // NS), ACC1 // NS)])


def _flush0(cid, sid, acc, out_hbm):
    @pl.when(sid < NS - 1)
    def _():
        pltpu.sync_copy(acc.at[pl.ds(sid * 632, 632)],
                        out_hbm.at[pl.ds(cid * HALF0 + sid * 632, 632)])

    @pl.when(sid == NS - 1)
    def _():
        pltpu.sync_copy(acc.at[pl.ds((NS - 1) * 632, 520)],
                        out_hbm.at[pl.ds(cid * HALF0 + (NS - 1) * 632, 520)])


def _flush1(cid, sid, acc, out_hbm):
    frows = ACC1 // NS
    pltpu.sync_copy(acc.at[pl.ds(sid * frows, frows)],
                    out_hbm.at[pl.ds(cid * ACC1 + sid * frows, frows)])


_agg0_body = _make_agg_body(
    rowfn=lambda cid, sid, j: sid + NS * j,
    validfn=lambda cid, sid, j: sid + NS * j < ROWS0,
    remapfn=_remap0,
    flushfn=_flush0,
    zerofn=_zero0,
    iters=ITER0,
)

_agg1_body = _make_agg_body(
    rowfn=lambda cid, sid, j: cid * (ROWS1 // NC) + sid + NS * j,
    validfn=lambda cid, sid, j: j < ITER1,
    remapfn=lambda cid, d: d,
    flushfn=_flush1,
    zerofn=_zero1,
    iters=ITER1,
)


def _fdot(a, w_ref):
    return jnp.dot(a, w_ref[...], preferred_element_type=jnp.float32)


def _mlp_body(a_ref, x_ref, w1_ref, b1_ref, w2_ref, b2_ref, o_ref):
    h = a_ref[...] + x_ref[...]
    h = jnp.maximum(_fdot(h, w1_ref) + b1_ref[...], 0.0)
    o_ref[...] = jnp.maximum(_fdot(h, w2_ref) + b2_ref[...], 0.0)


def _head_body(p0_ref, p1_ref, x_ref, w1_ref, b1_ref, w2_ref, b2_ref,
               l1_ref, c1_ref, l2_ref, c2_ref, o_ref):
    h = p0_ref[...] + p1_ref[...] + x_ref[...]
    h = jnp.maximum(_fdot(h, w1_ref) + b1_ref[...], 0.0)
    h = jnp.maximum(_fdot(h, w2_ref) + b2_ref[...], 0.0)
    h = jnp.maximum(_fdot(h, l1_ref) + c1_ref[...], 0.0)
    z = _fdot(h, l2_ref) + c2_ref[...]
    m = jnp.max(z, axis=-1, keepdims=True)
    e = jnp.exp(z - m)
    s = jnp.sum(e, axis=-1, keepdims=True)
    o_ref[...] = z - m - jnp.log(s)


def _full(shape):
    return pl.BlockSpec(shape, lambda i: (0, 0))


def _row(block, width=D, off=0):
    return pl.BlockSpec((block, width), lambda i, o=off: (i + o, 0))


def _mlp(aggr, x_full, w1, b1, w2, b2, rows, block):
    return pl.pallas_call(
        _mlp_body,
        grid=(rows // block,),
        in_specs=[_row(block), _row(block), _full((D, D)), _full((1, D)),
                  _full((D, D)), _full((1, D))],
        out_specs=_row(block),
        out_shape=jax.ShapeDtypeStruct((rows, D), jnp.float32),
    )(aggr, x_full, w1, b1, w2, b2)


def _head(parts, h_full, w1, b1, w2, b2, l1, c1, l2, c2, rows, block, dout):
    return pl.pallas_call(
        _head_body,
        grid=(rows // block,),
        in_specs=[_row(block), _row(block, off=N2 // block), _row(block),
                  _full((D, D)), _full((1, D)), _full((D, D)), _full((1, D)),
                  _full((D, D)), _full((1, D)), _full((D, dout)), _full((1, dout))],
        out_specs=pl.BlockSpec((block, dout), lambda i: (i, 0)),
        out_shape=jax.ShapeDtypeStruct((rows, dout), jnp.float32),
    )(parts, parts, h_full, w1, b1, w2, b2, l1, c1, l2, c2)


@jax.jit
def kernel(x, src0, dst0, src1, dst1, W0a, b0a, bn0_w, bn0_b, W0b, b0b,
           W1a, b1a, bn1_w, bn1_b, W1b, b1b, lin1_w, lin1_b, lin2_w, lin2_b):
    x = x.astype(jnp.float32)
    src0_2d = src0.astype(jnp.int32).reshape(ROWS0, CH)
    dst0_2d = dst0.astype(jnp.int32).reshape(ROWS0, CH)
    src1_2d = src1.astype(jnp.int32).reshape(ROWS1, CH)
    dst1_2d = dst1.astype(jnp.int32).reshape(ROWS1, CH)

    zeros0 = jnp.zeros((624, D), jnp.float32)
    zeros1 = jnp.zeros((ACC1 // NS, D), jnp.float32)

    s0 = bn0_w / jnp.sqrt(1.0 + BN_EPS)
    w0a = W0a.T * s0[None, :]
    c0a = (b0a * s0 + bn0_b).reshape(1, D)
    s1 = bn1_w / jnp.sqrt(1.0 + BN_EPS)
    w1a = W1a.T * s1[None, :]
    c1a = (b1a * s1 + bn1_b).reshape(1, D)
    dout = lin2_w.shape[0]

    agg0 = _agg_kernel(_agg0_body, N1, ACC0)
    aggr0 = agg0(x, src0_2d, dst0_2d, zeros0)
    h = _mlp(aggr0, x, w0a, c0a, W0b.T, b0b.reshape(1, D), N1, 2000)

    agg1 = _agg_kernel(_agg1_body, NC * N2, ACC1)
    parts = agg1(h, src1_2d, dst1_2d, zeros1)
    out = _head(parts, h, w1a, c1a, W1b.T, b1b.reshape(1, D),
                lin1_w.T, lin1_b.reshape(1, D), lin2_w.T,
                lin2_b.reshape(1, dout), N2, 1024, dout)
    return out
